# Initial kernel scaffold; baseline (speedup 1.0000x reference)
import jax
import jax.numpy as jnp
from jax.experimental import pallas as pl


def kernel(pos, tables, W1, W2, W3):
    raise NotImplementedError("write your pallas kernel here")


# speedup vs baseline: 232.1677x; 232.1677x over previous
import functools

import jax
import jax.numpy as jnp
import numpy as np
from jax import lax
from jax.experimental import pallas as pl
from jax.experimental.pallas import tpu as pltpu
from jax.experimental.pallas import tpu_sc as plsc

_N_LEVELS = 10
_F = 2
_T = 2 ** 14
_BASE_RES = 16
_SCALE = 1.5
_RES = [int(np.floor(_BASE_RES * _SCALE ** l)) for l in range(_N_LEVELS)]
_B = 262144
_D_IN = _N_LEVELS * _F

_P2 = np.uint32(2654435761)
_P3 = np.uint32(805459861)

_NC = 2
_NS = 16
_LANES = 16
_NW = _NC * _NS
_CHUNK = _B // _NW
_GROUPS = _CHUNK // _LANES


def _encode_body(x_hbm, y_hbm, z_hbm, tabs, out, x_v, y_v, z_v, tab_v, row_v):
    wid = lax.axis_index("s") * _NC + lax.axis_index("c")
    base = wid * _CHUNK
    pltpu.sync_copy(x_hbm.at[pl.ds(base, _CHUNK)], x_v)
    pltpu.sync_copy(y_hbm.at[pl.ds(base, _CHUNK)], y_v)
    pltpu.sync_copy(z_hbm.at[pl.ds(base, _CHUNK)], z_v)

    for l in range(_N_LEVELS):
        res = _RES[l]
        dense = (res + 1) ** 3 <= _T
        pltpu.sync_copy(tabs.at[pl.ds(l * _T * _F, _T * _F)], tab_v)

        def body(i, res=res, dense=dense):
            sl = pl.ds(i * _LANES, _LANES)
            x = x_v[sl]
            y = y_v[sl]
            z = z_v[sl]
            xs = x * float(res)
            ys = y * float(res)
            zs = z * float(res)
            xi = xs.astype(jnp.int32)
            yi = ys.astype(jnp.int32)
            zi = zs.astype(jnp.int32)
            fx = xs - xi.astype(jnp.float32)
            fy = ys - yi.astype(jnp.float32)
            fz = zs - zi.astype(jnp.float32)
            wx = (1.0 - fx, fx)
            wy = (1.0 - fy, fy)
            wz = (1.0 - fz, fz)

            if dense:
                s1 = res + 1
                b0 = (xi * s1 + yi) * s1 + zi
                idx2 = {}
                for dx in (0, 1):
                    for dy in (0, 1):
                        for dz in (0, 1):
                            off = dx * s1 * s1 + dy * s1 + dz
                            idx2[(dx, dy, dz)] = (b0 + off) * 2
            else:
                xu = xi.astype(jnp.uint32)
                yu = yi.astype(jnp.uint32)
                zu = zi.astype(jnp.uint32)
                hx = (xu, xu + jnp.uint32(1))
                hy0 = yu * _P2
                hy = (hy0, hy0 + _P2)
                hz0 = zu * _P3
                hz = (hz0, hz0 + _P3)
                mask = jnp.uint32(_T - 1)
                idx2 = {}
                for dx in (0, 1):
                    for dy in (0, 1):
                        for dz in (0, 1):
                            h = (hx[dx] ^ hy[dy] ^ hz[dz]) & mask
                            idx2[(dx, dy, dz)] = (h * jnp.uint32(2)).astype(jnp.int32)

            acc0 = jnp.zeros((_LANES,), jnp.float32)
            acc1 = jnp.zeros((_LANES,), jnp.float32)
            for dx in (0, 1):
                for dy in (0, 1):
                    wxy = wx[dx] * wy[dy]
                    for dz in (0, 1):
                        i2 = idx2[(dx, dy, dz)]
                        f0 = plsc.load_gather(tab_v, [i2])
                        f1 = plsc.load_gather(tab_v, [i2 + 1])
                        w = wxy * wz[dz]
                        acc0 = acc0 + f0 * w
                        acc1 = acc1 + f1 * w
            row_v[0, sl] = acc0
            row_v[1, sl] = acc1

        pl.loop(0, _GROUPS)(body)
        pltpu.sync_copy(row_v, out.at[pl.ds(2 * l, 2), pl.ds(base, _CHUNK)])


_enc_call = pl.kernel(
    _encode_body,
    out_type=jax.ShapeDtypeStruct((_D_IN, _B), jnp.float32),
    mesh=plsc.VectorSubcoreMesh(
        core_axis_name="c", subcore_axis_name="s", num_cores=_NC,
        num_subcores=_NS),
    scratch_types=[
        pltpu.VMEM((_CHUNK,), jnp.float32),
        pltpu.VMEM((_CHUNK,), jnp.float32),
        pltpu.VMEM((_CHUNK,), jnp.float32),
        pltpu.VMEM((_T * _F,), jnp.float32),
        pltpu.VMEM((2, _CHUNK), jnp.float32),
    ],
    compiler_params=pltpu.CompilerParams(needs_layout_passes=False),
)


_BLK = 4096


def _mlp_body(enc_ref, w1_ref, w2_ref, w3_ref, out_ref):
    e = enc_ref[...]
    h = jnp.dot(w1_ref[...], e, preferred_element_type=jnp.float32)
    h = jnp.maximum(h, 0.0)
    h = jnp.dot(w2_ref[...], h, preferred_element_type=jnp.float32)
    h = jnp.maximum(h, 0.0)
    o = jnp.dot(w3_ref[...], h, preferred_element_type=jnp.float32)
    out_ref[...] = jnp.clip(o, 0.0, 1.0)


def _mlp_call(encT, w1t, w2t, w3t):
    return pl.pallas_call(
        _mlp_body,
        grid=(_B // _BLK,),
        in_specs=[
            pl.BlockSpec((_D_IN, _BLK), lambda i: (0, i)),
            pl.BlockSpec((64, _D_IN), lambda i: (0, 0)),
            pl.BlockSpec((64, 64), lambda i: (0, 0)),
            pl.BlockSpec((1, 64), lambda i: (0, 0)),
        ],
        out_specs=pl.BlockSpec((1, _BLK), lambda i: (0, i)),
        out_shape=jax.ShapeDtypeStruct((1, _B), jnp.float32),
    )(encT, w1t, w2t, w3t)


@jax.jit
def _impl(pos, tables, W1, W2, W3):
    x, y, z = pos[:, 0], pos[:, 1], pos[:, 2]
    tabs = tables.reshape(_N_LEVELS * _T * _F)
    encT = _enc_call(x, y, z, tabs)
    outT = _mlp_call(encT, W1.T, W2.T, W3.T)
    return outT.reshape(_B, 1)


def kernel(pos, tables, W1, W2, W3):
    return _impl(pos, tables, W1, W2, W3)
